# Initial kernel scaffold; baseline (speedup 1.0000x reference)
import jax
import jax.numpy as jnp
from jax.experimental import pallas as pl


def kernel(x, mapping_index, W1, b1, W2, b2):
    raise NotImplementedError("write your pallas kernel here")


# speedup vs baseline: 6.0153x; 6.0153x over previous
import functools

import jax
import jax.numpy as jnp
from jax import lax
from jax.experimental import pallas as pl
from jax.experimental.pallas import tpu as pltpu
from jax.experimental.pallas import tpu_sc as plsc

N_NODES = 10000
NUM_HOM = 320000
KERNELS = 6
DIM = 128

_NC = 2
_NS = 16
_NW = _NC * _NS
_EPW = NUM_HOM // _NW
_CH = 40
_KS = 10
_NST = _EPW // (_CH * _KS)
_RPT = N_NODES // _NS


def _mlp_body(x_ref, w1_ref, b1_ref, w2_ref, b2_ref, y_ref):
    h = jnp.dot(x_ref[...], w1_ref[0], preferred_element_type=jnp.float32)
    h = jnp.maximum(h + b1_ref[0], 0.0)
    y_ref[0] = jnp.dot(h, w2_ref[0], preferred_element_type=jnp.float32) + b2_ref[0]


def _mlp_all(x, W1, b1, W2, b2):
    blk = 1000
    grid = (KERNELS, N_NODES // blk)
    return pl.pallas_call(
        _mlp_body,
        grid=grid,
        in_specs=[
            pl.BlockSpec((blk, DIM), lambda i, j: (j, 0)),
            pl.BlockSpec((1, DIM, DIM), lambda i, j: (i, 0, 0)),
            pl.BlockSpec((1, 1, DIM), lambda i, j: (i, 0, 0)),
            pl.BlockSpec((1, DIM, DIM), lambda i, j: (i, 0, 0)),
            pl.BlockSpec((1, 1, DIM), lambda i, j: (i, 0, 0)),
        ],
        out_specs=pl.BlockSpec((1, blk, DIM), lambda i, j: (i, j, 0)),
        out_shape=jax.ShapeDtypeStruct((KERNELS, N_NODES, DIM), jnp.float32),
    )(x, W1, b1.reshape(KERNELS, 1, DIM), W2, b2.reshape(KERNELS, 1, DIM))


def _sc_body(y_hbm, idx_hbm, out_hbm,
             gbuf_v, r0, r1, r2, r3, r4, r5, prod_v, accum_sh, sem):
    c = lax.axis_index("c")
    s = lax.axis_index("s")
    wid = c * _NS + s
    rows = (r0, r1, r2, r3, r4, r5)

    def _zp(e, carry):
        for k in range(DIM // 16):
            prod_v[e, pl.ds(k * 16, 16)] = jnp.zeros((16,), jnp.float32)
        return carry

    lax.fori_loop(0, _CH, _zp, 0)
    for r in range(_RPT // _CH):
        pltpu.sync_copy(prod_v, accum_sh.at[pl.ds(s * _RPT + r * _CH, _CH)])
    pltpu.sync_copy(
        prod_v.at[pl.ds(0, _RPT % _CH)],
        accum_sh.at[pl.ds(s * _RPT + (_RPT // _CH) * _CH, _RPT % _CH)],
    )
    plsc.subcore_barrier()

    cbase = wid * (_EPW // _CH)

    def _stage(ts, carry):
        sb = cbase + ts * _KS
        for i in range(KERNELS):
            pltpu.sync_copy(idx_hbm.at[i].at[pl.ds(sb, _KS)], gbuf_v.at[i])

        def _chunk(k, carry2):
            cps = [
                pltpu.async_copy(y_hbm.at[i].at[gbuf_v.at[i].at[k]], rows[i], sem)
                for i in range(KERNELS)
            ]
            for cp in cps:
                cp.wait()

            def _prod(e, carry3):
                for q in range(DIM // 16):
                    sl = pl.ds(q * 16, 16)
                    v = r0[e, sl] * r1[e, sl]
                    v = v * r2[e, sl]
                    v = v * r3[e, sl]
                    v = v * r4[e, sl]
                    v = v * r5[e, sl]
                    prod_v[e, sl] = v
                return carry3

            lax.fori_loop(0, _CH, _prod, 0)
            pltpu.sync_copy(prod_v, accum_sh.at[gbuf_v.at[0].at[k]], add=True)
            return carry2

        lax.fori_loop(0, _KS, _chunk, 0)
        return carry

    lax.fori_loop(0, _NST, _stage, 0)
    plsc.subcore_barrier()
    pltpu.sync_copy(
        accum_sh.at[pl.ds(s * _RPT, _RPT)],
        out_hbm.at[c].at[pl.ds(s * _RPT, _RPT)],
    )


@functools.cache
def _sc_kernel():
    return pl.kernel(
        _sc_body,
        mesh=plsc.VectorSubcoreMesh(core_axis_name="c", subcore_axis_name="s"),
        compiler_params=pltpu.CompilerParams(use_tc_tiling_on_sc=False),
        out_type=jax.ShapeDtypeStruct((_NC, N_NODES, DIM), jnp.float32),
        scratch_types=[
            pltpu.VMEM((KERNELS, _KS, _CH), jnp.int32),
        ] + [pltpu.VMEM((_CH, DIM), jnp.float32) for _ in range(KERNELS)] + [
            pltpu.VMEM((_CH, DIM), jnp.float32),
            pltpu.VMEM_SHARED((N_NODES, DIM), jnp.float32),
            pltpu.SemaphoreType.DMA,
        ],
    )


def _add_body(p_ref, o_ref):
    o_ref[...] = p_ref[0] + p_ref[1]


def _add_partials(partials):
    blk = 1000
    return pl.pallas_call(
        _add_body,
        grid=(N_NODES // blk,),
        in_specs=[pl.BlockSpec((_NC, blk, DIM), lambda j: (0, j, 0))],
        out_specs=pl.BlockSpec((blk, DIM), lambda j: (j, 0)),
        out_shape=jax.ShapeDtypeStruct((N_NODES, DIM), jnp.float32),
    )(partials)


def kernel(x, mapping_index, W1, b1, W2, b2):
    idx = mapping_index.astype(jnp.int32).T.reshape(KERNELS, NUM_HOM // _CH, _CH)
    y = _mlp_all(x, W1, b1, W2, b2)
    partials = _sc_kernel()(y, idx)
    return _add_partials(partials)


# speedup vs baseline: 7.1378x; 1.1866x over previous
import functools

import jax
import jax.numpy as jnp
from jax import lax
from jax.experimental import pallas as pl
from jax.experimental.pallas import tpu as pltpu
from jax.experimental.pallas import tpu_sc as plsc

N_NODES = 10000
NUM_HOM = 320000
KERNELS = 6
DIM = 128

_NC = 2
_NS = 16
_NW = _NC * _NS
_EPW = NUM_HOM // _NW
_CH = 40
_KS = 10
_NST = _EPW // (_CH * _KS)
_RPT = N_NODES // _NS


def _mlp_body(x_ref, w1_ref, b1_ref, w2_ref, b2_ref, y_ref):
    h = jnp.dot(x_ref[...], w1_ref[0], preferred_element_type=jnp.float32)
    h = jnp.maximum(h + b1_ref[0], 0.0)
    y_ref[0] = jnp.dot(h, w2_ref[0], preferred_element_type=jnp.float32) + b2_ref[0]


def _mlp_all(x, W1, b1, W2, b2):
    blk = 1000
    grid = (KERNELS, N_NODES // blk)
    return pl.pallas_call(
        _mlp_body,
        grid=grid,
        in_specs=[
            pl.BlockSpec((blk, DIM), lambda i, j: (j, 0)),
            pl.BlockSpec((1, DIM, DIM), lambda i, j: (i, 0, 0)),
            pl.BlockSpec((1, 1, DIM), lambda i, j: (i, 0, 0)),
            pl.BlockSpec((1, DIM, DIM), lambda i, j: (i, 0, 0)),
            pl.BlockSpec((1, 1, DIM), lambda i, j: (i, 0, 0)),
        ],
        out_specs=pl.BlockSpec((1, blk, DIM), lambda i, j: (i, j, 0)),
        out_shape=jax.ShapeDtypeStruct((KERNELS, N_NODES, DIM), jnp.float32),
    )(x, W1, b1.reshape(KERNELS, 1, DIM), W2, b2.reshape(KERNELS, 1, DIM))


def _lo(u):
    return lax.bitcast_convert_type(u << 16, jnp.float32)


def _hi(u):
    return lax.bitcast_convert_type(u & jnp.int32(-65536), jnp.float32)


def _sc_body(y_hbm, idx_hbm, out_hbm, gbuf_v,
             a0, a1, a2, a3, a4, a5, b0, b1, b2, b3, b4, b5,
             prod_v, accum_sh, sem_a, sem_b):
    c = lax.axis_index("c")
    s = lax.axis_index("s")
    wid = c * _NS + s
    bufs = ((a0, a1, a2, a3, a4, a5), (b0, b1, b2, b3, b4, b5))
    sems = (sem_a, sem_b)

    def _zp(e, carry):
        for k in range(DIM // 16):
            prod_v[e, pl.ds(k * 16, 16)] = jnp.zeros((16,), jnp.float32)
        return carry

    lax.fori_loop(0, _CH, _zp, 0)
    for r in range(_RPT // _CH):
        pltpu.sync_copy(prod_v, accum_sh.at[pl.ds(s * _RPT + r * _CH, _CH)])
    pltpu.sync_copy(
        prod_v.at[pl.ds(0, _RPT % _CH)],
        accum_sh.at[pl.ds(s * _RPT + (_RPT // _CH) * _CH, _RPT % _CH)],
    )
    plsc.subcore_barrier()

    cbase = wid * (_EPW // _CH)

    def _fire(k, pick):
        return [
            pltpu.async_copy(
                y_hbm.at[i].at[gbuf_v.at[i].at[k]], bufs[pick][i], sems[pick]
            )
            for i in range(KERNELS)
        ]

    def _stage(ts, carry):
        sb = cbase + ts * _KS
        for i in range(KERNELS):
            pltpu.sync_copy(idx_hbm.at[i].at[pl.ds(sb, _KS)], gbuf_v.at[i])

        cps = {0: _fire(0, 0)}
        for k in range(_KS):
            pick = k % 2
            if k + 1 < _KS:
                cps[k + 1] = _fire(k + 1, 1 - pick)
            for cp in cps.pop(k):
                cp.wait()
            rows = bufs[pick]

            def _prod(e, carry2):
                for q in range(DIM // 32):
                    sl = pl.ds(q * 16, 16)
                    us = [rows[i][e, sl] for i in range(KERNELS)]
                    lo = _lo(us[0])
                    hi = _hi(us[0])
                    for i in range(1, KERNELS):
                        lo = lo * _lo(us[i])
                        hi = hi * _hi(us[i])
                    prod_v[e, pl.ds(q * 32, 16)] = lo
                    prod_v[e, pl.ds(q * 32 + 16, 16)] = hi
                return carry2

            lax.fori_loop(0, _CH, _prod, 0)
            pltpu.sync_copy(prod_v, accum_sh.at[gbuf_v.at[0].at[k]], add=True)
        return carry

    lax.fori_loop(0, _NST, _stage, 0)
    plsc.subcore_barrier()
    pltpu.sync_copy(
        accum_sh.at[pl.ds(s * _RPT, _RPT)],
        out_hbm.at[c].at[pl.ds(s * _RPT, _RPT)],
    )


@functools.cache
def _sc_kernel():
    return pl.kernel(
        _sc_body,
        mesh=plsc.VectorSubcoreMesh(core_axis_name="c", subcore_axis_name="s"),
        compiler_params=pltpu.CompilerParams(use_tc_tiling_on_sc=False),
        out_type=jax.ShapeDtypeStruct((_NC, N_NODES, DIM), jnp.float32),
        scratch_types=[
            pltpu.VMEM((KERNELS, _KS, _CH), jnp.int32),
        ] + [pltpu.VMEM((_CH, DIM // 2), jnp.int32) for _ in range(2 * KERNELS)] + [
            pltpu.VMEM((_CH, DIM), jnp.float32),
            pltpu.VMEM_SHARED((N_NODES, DIM), jnp.float32),
            pltpu.SemaphoreType.DMA,
            pltpu.SemaphoreType.DMA,
        ],
    )


def _add_body(p_ref, o_ref):
    o_ref[...] = p_ref[0] + p_ref[1]


def _add_partials(partials):
    blk = 1000
    return pl.pallas_call(
        _add_body,
        grid=(N_NODES // blk,),
        in_specs=[pl.BlockSpec((_NC, blk, DIM), lambda j: (0, j, 0))],
        out_specs=pl.BlockSpec((blk, DIM), lambda j: (j, 0)),
        out_shape=jax.ShapeDtypeStruct((N_NODES, DIM), jnp.float32),
    )(partials)


def kernel(x, mapping_index, W1, b1, W2, b2):
    idx = mapping_index.astype(jnp.int32).T.reshape(KERNELS, NUM_HOM // _CH, _CH)
    y = _mlp_all(x, W1, b1, W2, b2)
    ybf = (
        y.reshape(KERNELS, N_NODES, DIM // 32, 2, 16)
        .transpose(0, 1, 2, 4, 3)
        .reshape(KERNELS, N_NODES, DIM // 2, 2)
        .astype(jnp.bfloat16)
    )
    yi = lax.bitcast_convert_type(ybf, jnp.int32)
    partials = _sc_kernel()(yi, idx)
    return _add_partials(partials)
